# Initial kernel scaffold; baseline (speedup 1.0000x reference)
import jax
import jax.numpy as jnp
from jax.experimental import pallas as pl


def kernel(e_map, v_count, Y, W):
    raise NotImplementedError("write your pallas kernel here")


# speedup vs baseline: 108.8849x; 108.8849x over previous
import functools

import jax
import jax.numpy as jnp
from jax import lax
from jax.experimental import pallas as pl
from jax.experimental.pallas import tpu as pltpu
from jax.experimental.pallas import tpu_sc as plsc

N_SEG = 100000
M_ELEM = 6400000
K = 16
NW = 32
SEG_PER_W = 3128
SEG_LAST = N_SEG - (NW - 1) * SEG_PER_W
YLOC = 3136
CHUNK = 8192
NB = 48

def _merge_top16(top, piece):
    ps, _ = plsc.sort_key_val(piece, piece, descending=True)
    m = jnp.maximum(top, jnp.flip(ps, 0))
    ts, _ = plsc.sort_key_val(m, m, descending=True)
    return ts


def _negv():
    return jnp.full((K,), -jnp.inf, dtype=jnp.float32)


def _sort_pool_body(e_hbm, yv_hbm, b_hbm, wk_hbm, out_hbm,
                    ids_v, vals_v, yloc, b_v, wk_v, tmp_ids):
    wid = lax.axis_index("c") * 16 + lax.axis_index("s")
    pltpu.sync_copy(b_hbm, b_v)
    pltpu.sync_copy(wk_hbm, wk_v)
    wk = wk_v[...]
    bwin = b_v[pl.ds(wid, K)]
    start = bwin[0]
    end = bwin[1]
    s0 = wid * SEG_PER_W
    s1m1 = jnp.minimum(s0 + SEG_PER_W, N_SEG) - 1
    iota = lax.iota(jnp.int32, K)
    lane0 = iota == 0

    def zero_body(i, c):
        yloc[pl.ds(i * K, K)] = jnp.zeros((K,), jnp.float32)
        return c

    lax.fori_loop(0, YLOC // K, zero_body, 0)

    def flush_val(top):
        return jnp.sum(jnp.where(top == _negv(), 0.0, top) * wk)

    start8 = start // 8 * 8
    n_chunks = (end - start8 + CHUNK - 1) // CHUNK

    def chunk_body(k, carry):
        nominal = start8 + k * CHUNK
        cstart = jnp.minimum(nominal, M_ELEM - CHUNK)
        pltpu.sync_copy(e_hbm.at[pl.ds(cstart, CHUNK)], ids_v)
        pltpu.sync_copy(yv_hbm.at[pl.ds(cstart, CHUNK)], vals_v)
        lo = jnp.maximum(start, nominal)

        def vec_body(v, c2):
            cur, top = c2
            ids_raw = ids_v[pl.ds(v * K, K)]
            vals_raw = vals_v[pl.ds(v * K, K)]
            gidx = cstart + v * K + iota
            valid = (gidx >= lo) & (gidx < end)
            vals_m = jnp.where(valid, vals_raw, -jnp.inf)
            ids_fill = jnp.where(valid, ids_raw, cur).astype(jnp.float32)
            srt, _ = plsc.sort_key_val(ids_fill, ids_fill, descending=True)
            mx = srt[0].astype(jnp.int32)
            ids_t = jnp.where(valid, ids_raw, jnp.where(gidx < lo, cur, mx))
            tmp_ids[pl.ds(0, K)] = ids_t

            def wcond(st):
                return st[0] < K

            def wbody(st):
                pos, cur, top = st
                seg = tmp_ids[pl.ds(pos, K)][0]
                mask = ids_t == seg
                n = jnp.sum(mask.astype(jnp.int32))
                piece = jnp.where(mask, vals_m, -jnp.inf)
                do_flush = seg != cur
                contrib = flush_val(top)
                plsc.store_scatter(
                    yloc,
                    (jnp.full((K,), cur - s0, jnp.int32),),
                    jnp.full((K,), contrib, jnp.float32),
                    mask=lane0 & do_flush,
                )
                top = jnp.where(do_flush, _negv(), top)
                cur = jnp.where(do_flush, seg, cur)
                top = _merge_top16(top, piece)
                return pos + n, cur, top

            _, cur, top = lax.while_loop(wcond, wbody, (jnp.int32(0), cur, top))
            return cur, top

        return lax.fori_loop(0, CHUNK // K, vec_body, carry)

    cur, top = lax.fori_loop(0, n_chunks, chunk_body, (s0, _negv()))
    plsc.store_scatter(
        yloc,
        (jnp.full((K,), cur - s0, jnp.int32),),
        jnp.full((K,), flush_val(top), jnp.float32),
        mask=lane0,
    )

    @pl.when(wid != NW - 1)
    def _():
        pltpu.sync_copy(yloc.at[pl.ds(0, SEG_PER_W)],
                        out_hbm.at[pl.ds(s0, SEG_PER_W)])

    @pl.when(wid == NW - 1)
    def _():
        pltpu.sync_copy(yloc.at[pl.ds(0, SEG_LAST)],
                        out_hbm.at[pl.ds(s0, SEG_LAST)])


@functools.lru_cache(maxsize=1)
def _build_sc_kernel():
    mesh = plsc.VectorSubcoreMesh(core_axis_name="c", subcore_axis_name="s")
    return pl.kernel(
        _sort_pool_body,
        out_type=jax.ShapeDtypeStruct((N_SEG,), jnp.float32),
        mesh=mesh,
        compiler_params=pltpu.CompilerParams(needs_layout_passes=False),
        scratch_types=[
            pltpu.VMEM((CHUNK,), jnp.int32),
            pltpu.VMEM((CHUNK,), jnp.float32),
            pltpu.VMEM((YLOC,), jnp.float32),
            pltpu.VMEM((NB,), jnp.int32),
            pltpu.VMEM((K,), jnp.float32),
            pltpu.VMEM((2 * K,), jnp.int32),
        ],
    )


def kernel(e_map, v_count, Y, W):
    del v_count
    yf = jnp.squeeze(Y, -1)
    wk = jax.nn.softmax(W.astype(jnp.float32))
    seg_starts = jnp.arange(NW + 1, dtype=jnp.int32) * SEG_PER_W
    bounds = jnp.searchsorted(e_map, seg_starts).astype(jnp.int32)
    bounds = jnp.zeros((NB,), jnp.int32).at[: NW + 1].set(bounds)
    return _build_sc_kernel()(e_map, yf, bounds, wk)
